# baseline (speedup 1.0000x reference)
import functools

import numpy as np
import jax
import jax.numpy as jnp
from jax import lax
from jax.experimental import pallas as pl
from jax.experimental.pallas import tpu as pltpu
from jax.experimental.pallas import tpu_sc as plsc

N = 10000
E = 160000
W = 32
DEPTH = 2

NC = 2
NS = 16
NWK = NC * NS
CHUNK = 128
NCH = E // CHUNK
CBASE = NCH // NWK
CEXTRA = NCH - CBASE * NWK
CPW = CBASE + 1
NBUF = 4
NP = 10240
TROWS = NP // NS

G_MAT = np.repeat(np.eye(W, dtype=np.float32), W, axis=0)

BLK_E = 3200
BLK_N = 1000

_MESH = plsc.VectorSubcoreMesh(core_axis_name="c", subcore_axis_name="s")
_SC_PARAMS = pltpu.CompilerParams(use_tc_tiling_on_sc=False)


def _stage_idx(idx_hbm, idx_v, wid):
    pltpu.sync_copy(idx_hbm.at[pl.ds(wid * CBASE, CBASE)],
                    idx_v.at[pl.ds(0, CBASE)])

    @pl.when(wid < CEXTRA)
    def _():
        pltpu.sync_copy(idx_hbm.at[CBASE * NWK + wid], idx_v.at[CBASE])


def _chunk_of(j, wid):
    return jnp.where(j < CBASE, wid * CBASE + j, CBASE * NWK + wid)


@functools.partial(
    pl.kernel,
    mesh=_MESH,
    compiler_params=_SC_PARAMS,
    out_type=jax.ShapeDtypeStruct((E, W), jnp.bfloat16),
    scratch_types=[
        pltpu.VMEM((CPW, CHUNK), jnp.int32),
        [pltpu.VMEM((CHUNK, W), jnp.bfloat16) for _ in range(NBUF)],
        [pltpu.SemaphoreType.DMA for _ in range(NBUF)],
    ],
)
def _sc_gather(h_hbm, idx_hbm, out_hbm, idx_v, rows, sems):
    wid = lax.axis_index("c") * NS + lax.axis_index("s")
    cnt = jnp.where(wid < CEXTRA, CBASE + 1, CBASE)
    _stage_idx(idx_hbm, idx_v, wid)

    def body(t, c):
        for b in range(NBUF):
            j = t * NBUF + b

            @pl.when(j < cnt)
            def _(b=b, j=j):
                pltpu.async_copy(h_hbm.at[idx_v.at[j]], rows[b], sems[b])

        for b in range(NBUF):
            j = t * NBUF + b

            @pl.when(j < cnt)
            def _(b=b, j=j):
                pltpu.make_async_copy(h_hbm.at[idx_v.at[j]], rows[b],
                                      sems[b]).wait()
                pltpu.sync_copy(
                    rows[b],
                    out_hbm.at[pl.ds(_chunk_of(j, wid) * CHUNK, CHUNK)])

        return c

    lax.fori_loop(0, -(-CPW // NBUF), body, 0)


def _make_sc_scatter(with_deg):
    out_types = [jax.ShapeDtypeStruct((NC, NP, W), jnp.float32)]
    scratch = [
        pltpu.VMEM((CPW, CHUNK), jnp.int32),
        [pltpu.VMEM((CHUNK, W), jnp.float32) for _ in range(NBUF)],
        [pltpu.SemaphoreType.DMA for _ in range(NBUF)],
        pltpu.VMEM((CHUNK, W), jnp.float32),
        pltpu.VMEM((TROWS, W), jnp.float32),
        pltpu.VMEM_SHARED((NP, W), jnp.float32),
    ]
    if with_deg:
        out_types.append(jax.ShapeDtypeStruct((NC, NP, W), jnp.float32))
        scratch += [
            pltpu.VMEM((CHUNK, W), jnp.float32),
            pltpu.VMEM_SHARED((NP, W), jnp.float32),
        ]

    def body_fn(msg_hbm, idx_hbm, *rest):
        if with_deg:
            (s_out, d_out, idx_v, rows, sems, zeros_v, stage_v, acc,
             ones_v, dacc) = rest
        else:
            (s_out, idx_v, rows, sems, zeros_v, stage_v, acc) = rest
        cid = lax.axis_index("c")
        sid = lax.axis_index("s")
        wid = cid * NS + sid
        cnt = jnp.where(wid < CEXTRA, CBASE + 1, CBASE)
        base = sid * TROWS

        _stage_idx(idx_hbm, idx_v, wid)

        z16 = jnp.zeros((16,), jnp.float32)
        o16 = jnp.ones((16,), jnp.float32)

        def fill(i, c):
            zeros_v[i, pl.ds(0, 16)] = z16
            zeros_v[i, pl.ds(16, 16)] = z16
            if with_deg:
                ones_v[i, pl.ds(0, 16)] = o16
                ones_v[i, pl.ds(16, 16)] = o16
            return c

        lax.fori_loop(0, CHUNK, fill, 0)

        def zero_acc(k, c):
            pltpu.sync_copy(zeros_v, acc.at[pl.ds(base + k * CHUNK, CHUNK)])
            if with_deg:
                pltpu.sync_copy(zeros_v, dacc.at[pl.ds(base + k * CHUNK, CHUNK)])
            return c

        lax.fori_loop(0, TROWS // CHUNK, zero_acc, 0)
        plsc.subcore_barrier()

        def body(t, c):
            for b in range(NBUF):
                j = t * NBUF + b

                @pl.when(j < cnt)
                def _(b=b, j=j):
                    pltpu.async_copy(
                        msg_hbm.at[pl.ds(_chunk_of(j, wid) * CHUNK, CHUNK)],
                        rows[b], sems[b])

            for b in range(NBUF):
                j = t * NBUF + b

                @pl.when(j < cnt)
                def _(b=b, j=j):
                    pltpu.make_async_copy(
                        msg_hbm.at[pl.ds(_chunk_of(j, wid) * CHUNK, CHUNK)],
                        rows[b], sems[b]).wait()
                    pltpu.sync_copy(rows[b], acc.at[idx_v.at[j]], add=True)
                    if with_deg:
                        pltpu.sync_copy(ones_v, dacc.at[idx_v.at[j]], add=True)

            return c

        lax.fori_loop(0, -(-CPW // NBUF), body, 0)
        plsc.subcore_barrier()

        pltpu.sync_copy(acc.at[pl.ds(base, TROWS)], stage_v)
        pltpu.sync_copy(stage_v, s_out.at[cid, pl.ds(base, TROWS)])
        if with_deg:
            pltpu.sync_copy(dacc.at[pl.ds(base, TROWS)], stage_v)
            pltpu.sync_copy(stage_v, d_out.at[cid, pl.ds(base, TROWS)])

    return pl.kernel(
        body_fn,
        mesh=_MESH,
        compiler_params=_SC_PARAMS,
        out_type=tuple(out_types),
        scratch_types=scratch,
    )


_sc_scatter_deg = _make_sc_scatter(True)
_sc_scatter = _make_sc_scatter(False)


def _fc1_body(x_ref, w_ref, b_ref, o_ref, ob_ref):
    h = x_ref[...] * w_ref[...] + b_ref[...]
    o_ref[...] = h
    ob_ref[...] = h.astype(jnp.bfloat16)


def _tc_fc1(x, fc1_W, fc1_b):
    return pl.pallas_call(
        _fc1_body,
        grid=(N // BLK_N,),
        in_specs=[
            pl.BlockSpec((BLK_N, 1), lambda i: (i, 0)),
            pl.BlockSpec((1, W), lambda i: (0, 0)),
            pl.BlockSpec((1, W), lambda i: (0, 0)),
        ],
        out_specs=[
            pl.BlockSpec((BLK_N, W), lambda i: (i, 0)),
            pl.BlockSpec((BLK_N, W), lambda i: (i, 0)),
        ],
        out_shape=[
            jax.ShapeDtypeStruct((N, W), jnp.float32),
            jax.ShapeDtypeStruct((N, W), jnp.bfloat16),
        ],
    )(x, fc1_W, fc1_b.reshape(1, W))


def _msg_body(ea, xj, w0, b0, w1, b1, w2, b2, gm, out):
    f32 = jnp.float32
    bf16 = jnp.bfloat16
    t = jnp.maximum(jnp.dot(ea[...].astype(bf16), w0[...],
                            preferred_element_type=f32) + b0[...], 0.0)
    t = jnp.maximum(jnp.dot(t.astype(bf16), w1[...],
                            preferred_element_type=f32) + b1[...], 0.0)
    k2 = jnp.dot(t.astype(bf16), w2[...],
                 preferred_element_type=f32).astype(bf16) + b2[...]
    xe = jnp.concatenate([xj[...]] * W, axis=1)
    out[...] = jnp.dot(k2 * xe, gm[...], preferred_element_type=f32)


def _tc_msg(edge_attr, xj, kW0, kb0, kW1, kb1, kW2, kb2, gm):
    rep = lambda i: (0, 0)
    return pl.pallas_call(
        _msg_body,
        grid=(E // BLK_E,),
        in_specs=[
            pl.BlockSpec((BLK_E, 6), lambda i: (i, 0)),
            pl.BlockSpec((BLK_E, W), lambda i: (i, 0)),
            pl.BlockSpec((6, 64), rep),
            pl.BlockSpec((1, 64), rep),
            pl.BlockSpec((64, 128), rep),
            pl.BlockSpec((1, 128), rep),
            pl.BlockSpec((128, 1024), rep),
            pl.BlockSpec((1, 1024), rep),
            pl.BlockSpec((1024, W), rep),
        ],
        out_specs=pl.BlockSpec((BLK_E, W), lambda i: (i, 0)),
        out_shape=jax.ShapeDtypeStruct((E, W), jnp.float32),
    )(edge_attr, xj, kW0, kb0.reshape(1, -1), kW1, kb1.reshape(1, -1),
      kW2, kb2.reshape(1, -1), gm)


def _comb_body(h_ref, s0, s1, d0, d1, w0, b0, w1, b1, w2, b2, gm,
               fw, fb, oh, ohb, oy):
    f32 = jnp.float32
    bf16 = jnp.bfloat16
    hb = h_ref[...]
    t = jnp.maximum(jnp.dot(hb.astype(bf16), w0[...],
                            preferred_element_type=f32) + b0[...], 0.0)
    t = jnp.maximum(jnp.dot(t.astype(bf16), w1[...],
                            preferred_element_type=f32) + b1[...], 0.0)
    bx = jnp.dot(t.astype(bf16), w2[...],
                 preferred_element_type=f32).astype(bf16) + b2[...]
    he = jnp.concatenate([hb.astype(bf16)] * W, axis=1)
    nl = jnp.dot(he * bx, gm[...], preferred_element_type=f32)
    deg = jnp.maximum(d0[...][0] + d1[...][0], 1.0)[:, 0:1]
    s = (s0[...][0] + s1[...][0]) / deg
    hn = 0.5 * s + 0.5 * (nl - hb) + hb
    oh[...] = hn
    ohb[...] = hn.astype(bf16)
    oy[...] = jnp.dot(hn, fw[...], preferred_element_type=f32) + fb[...]


def _tc_combine(h, sp, dp, bW0, bb0, bW1, bb1, bW2, bb2, gm,
                fc2_W, fc2_b):
    rep = lambda i: (0, 0)
    blk = lambda i: (i, 0)
    return pl.pallas_call(
        _comb_body,
        grid=(N // BLK_N,),
        in_specs=[
            pl.BlockSpec((BLK_N, W), blk),
            pl.BlockSpec((1, BLK_N, W), lambda i: (0, i, 0)),
            pl.BlockSpec((1, BLK_N, W), lambda i: (1, i, 0)),
            pl.BlockSpec((1, BLK_N, W), lambda i: (0, i, 0)),
            pl.BlockSpec((1, BLK_N, W), lambda i: (1, i, 0)),
            pl.BlockSpec((W, 64), rep),
            pl.BlockSpec((1, 64), rep),
            pl.BlockSpec((64, 128), rep),
            pl.BlockSpec((1, 128), rep),
            pl.BlockSpec((128, 1024), rep),
            pl.BlockSpec((1, 1024), rep),
            pl.BlockSpec((1024, W), rep),
            pl.BlockSpec((W, 1), rep),
            pl.BlockSpec((1, 1), rep),
        ],
        out_specs=[
            pl.BlockSpec((BLK_N, W), blk),
            pl.BlockSpec((BLK_N, W), blk),
            pl.BlockSpec((BLK_N, 1), blk),
        ],
        out_shape=[
            jax.ShapeDtypeStruct((N, W), jnp.float32),
            jax.ShapeDtypeStruct((N, W), jnp.bfloat16),
            jax.ShapeDtypeStruct((N, 1), jnp.float32),
        ],
    )(h, sp, sp, dp, dp, bW0, bb0.reshape(1, -1),
      bW1, bb1.reshape(1, -1), bW2, bb2.reshape(1, -1), gm,
      fc2_W, fc2_b.reshape(1, 1))


def kernel(x, edge_index, edge_attr, fc1_W, fc1_b, kW0, kb0, kW1, kb1,
           kW2, kb2, bW0, bb0, bW1, bb1, bW2, bb2, fc2_W, fc2_b):
    src2d = edge_index[0].reshape(NCH, CHUNK)
    dst2d = edge_index[1].reshape(NCH, CHUNK)
    bf16 = jnp.bfloat16
    gm = jnp.asarray(G_MAT, dtype=bf16)
    kW0b, kW1b, kW2b = kW0.astype(bf16), kW1.astype(bf16), kW2.astype(bf16)
    bW0b, bW1b, bW2b = bW0.astype(bf16), bW1.astype(bf16), bW2.astype(bf16)
    kb2b, bb2b = kb2.astype(bf16), bb2.astype(bf16)

    h, hb = _tc_fc1(x, fc1_W, fc1_b)
    dp = None
    out = None
    for d in range(DEPTH):
        xj = _sc_gather(hb, src2d)
        msg = _tc_msg(edge_attr, xj, kW0b, kb0, kW1b, kb1, kW2b, kb2b, gm)
        if d == 0:
            sp, dp = _sc_scatter_deg(msg, dst2d)
        else:
            (sp,) = _sc_scatter(msg, dst2d)
        h, hb, out = _tc_combine(h, sp, dp, bW0b, bb0, bW1b, bb1, bW2b, bb2b,
                                 gm, fc2_W, fc2_b)
    return out


# speedup vs baseline: 1.0515x; 1.0515x over previous
import functools

import numpy as np
import jax
import jax.numpy as jnp
from jax import lax
from jax.experimental import pallas as pl
from jax.experimental.pallas import tpu as pltpu
from jax.experimental.pallas import tpu_sc as plsc

N = 10000
E = 160000
W = 32
DEPTH = 2

NC = 2
NS = 16
NWK = NC * NS
CHUNK = 128
NCH = E // CHUNK
CBASE = NCH // NWK
CEXTRA = NCH - CBASE * NWK
CPW = CBASE + 1
NBUF = 8
NP = 10240
TROWS = NP // NS

G_MAT = np.repeat(np.eye(W, dtype=np.float32), W, axis=0)

BLK_E = 3200
BLK_N = 1000

_MESH = plsc.VectorSubcoreMesh(core_axis_name="c", subcore_axis_name="s")
_SC_PARAMS = pltpu.CompilerParams(use_tc_tiling_on_sc=False)


def _stage_idx(idx_hbm, idx_v, wid):
    pltpu.sync_copy(idx_hbm.at[pl.ds(wid * CBASE, CBASE)],
                    idx_v.at[pl.ds(0, CBASE)])

    @pl.when(wid < CEXTRA)
    def _():
        pltpu.sync_copy(idx_hbm.at[CBASE * NWK + wid], idx_v.at[CBASE])


def _chunk_of(j, wid):
    return jnp.where(j < CBASE, wid * CBASE + j, CBASE * NWK + wid)


@functools.partial(
    pl.kernel,
    mesh=_MESH,
    compiler_params=_SC_PARAMS,
    out_type=jax.ShapeDtypeStruct((E, W), jnp.float32),
    scratch_types=[
        pltpu.VMEM((CPW, CHUNK), jnp.int32),
        [pltpu.VMEM((CHUNK, W), jnp.float32) for _ in range(NBUF)],
        [pltpu.SemaphoreType.DMA for _ in range(NBUF)],
    ],
)
def _sc_gather(h_hbm, idx_hbm, out_hbm, idx_v, rows, sems):
    wid = lax.axis_index("c") * NS + lax.axis_index("s")
    cnt = jnp.where(wid < CEXTRA, CBASE + 1, CBASE)
    _stage_idx(idx_hbm, idx_v, wid)

    def body(t, c):
        for b in range(NBUF):
            j = t * NBUF + b

            @pl.when(j < cnt)
            def _(b=b, j=j):
                pltpu.async_copy(h_hbm.at[idx_v.at[j]], rows[b], sems[b])

        for b in range(NBUF):
            j = t * NBUF + b

            @pl.when(j < cnt)
            def _(b=b, j=j):
                pltpu.make_async_copy(h_hbm.at[idx_v.at[j]], rows[b],
                                      sems[b]).wait()
                pltpu.sync_copy(
                    rows[b],
                    out_hbm.at[pl.ds(_chunk_of(j, wid) * CHUNK, CHUNK)])

        return c

    lax.fori_loop(0, -(-CPW // NBUF), body, 0)


def _make_sc_scatter(with_deg):
    out_types = [jax.ShapeDtypeStruct((NC, NP, W), jnp.float32)]
    scratch = [
        pltpu.VMEM((CPW, CHUNK), jnp.int32),
        [pltpu.VMEM((CHUNK, W), jnp.float32) for _ in range(NBUF)],
        [pltpu.SemaphoreType.DMA for _ in range(NBUF)],
        pltpu.VMEM((CHUNK, W), jnp.float32),
        pltpu.VMEM((TROWS, W), jnp.float32),
        pltpu.VMEM_SHARED((NP, W), jnp.float32),
    ]
    if with_deg:
        out_types.append(jax.ShapeDtypeStruct((NC, NP, W), jnp.float32))
        scratch += [
            pltpu.VMEM((CHUNK, W), jnp.float32),
            pltpu.VMEM_SHARED((NP, W), jnp.float32),
        ]

    def body_fn(msg_hbm, idx_hbm, *rest):
        if with_deg:
            (s_out, d_out, idx_v, rows, sems, zeros_v, stage_v, acc,
             ones_v, dacc) = rest
        else:
            (s_out, idx_v, rows, sems, zeros_v, stage_v, acc) = rest
        cid = lax.axis_index("c")
        sid = lax.axis_index("s")
        wid = cid * NS + sid
        cnt = jnp.where(wid < CEXTRA, CBASE + 1, CBASE)
        base = sid * TROWS

        _stage_idx(idx_hbm, idx_v, wid)

        z16 = jnp.zeros((16,), jnp.float32)
        o16 = jnp.ones((16,), jnp.float32)

        def fill(i, c):
            zeros_v[i, pl.ds(0, 16)] = z16
            zeros_v[i, pl.ds(16, 16)] = z16
            if with_deg:
                ones_v[i, pl.ds(0, 16)] = o16
                ones_v[i, pl.ds(16, 16)] = o16
            return c

        lax.fori_loop(0, CHUNK, fill, 0)

        def zero_acc(k, c):
            pltpu.sync_copy(zeros_v, acc.at[pl.ds(base + k * CHUNK, CHUNK)])
            if with_deg:
                pltpu.sync_copy(zeros_v, dacc.at[pl.ds(base + k * CHUNK, CHUNK)])
            return c

        lax.fori_loop(0, TROWS // CHUNK, zero_acc, 0)
        plsc.subcore_barrier()

        def body(t, c):
            for b in range(NBUF):
                j = t * NBUF + b

                @pl.when(j < cnt)
                def _(b=b, j=j):
                    pltpu.async_copy(
                        msg_hbm.at[pl.ds(_chunk_of(j, wid) * CHUNK, CHUNK)],
                        rows[b], sems[b])

            for b in range(NBUF):
                j = t * NBUF + b

                @pl.when(j < cnt)
                def _(b=b, j=j):
                    pltpu.make_async_copy(
                        msg_hbm.at[pl.ds(_chunk_of(j, wid) * CHUNK, CHUNK)],
                        rows[b], sems[b]).wait()
                    pltpu.sync_copy(rows[b], acc.at[idx_v.at[j]], add=True)
                    if with_deg:
                        pltpu.sync_copy(ones_v, dacc.at[idx_v.at[j]], add=True)

            return c

        lax.fori_loop(0, -(-CPW // NBUF), body, 0)
        plsc.subcore_barrier()

        pltpu.sync_copy(acc.at[pl.ds(base, TROWS)], stage_v)
        pltpu.sync_copy(stage_v, s_out.at[cid, pl.ds(base, TROWS)])
        if with_deg:
            pltpu.sync_copy(dacc.at[pl.ds(base, TROWS)], stage_v)
            pltpu.sync_copy(stage_v, d_out.at[cid, pl.ds(base, TROWS)])

    return pl.kernel(
        body_fn,
        mesh=_MESH,
        compiler_params=_SC_PARAMS,
        out_type=tuple(out_types),
        scratch_types=scratch,
    )


_sc_scatter_deg = _make_sc_scatter(True)
_sc_scatter = _make_sc_scatter(False)


def _fc1_body(x_ref, w_ref, b_ref, o_ref):
    o_ref[...] = x_ref[...] * w_ref[...] + b_ref[...]


def _tc_fc1(x, fc1_W, fc1_b):
    return pl.pallas_call(
        _fc1_body,
        grid=(N // BLK_N,),
        in_specs=[
            pl.BlockSpec((BLK_N, 1), lambda i: (i, 0)),
            pl.BlockSpec((1, W), lambda i: (0, 0)),
            pl.BlockSpec((1, W), lambda i: (0, 0)),
        ],
        out_specs=pl.BlockSpec((BLK_N, W), lambda i: (i, 0)),
        out_shape=jax.ShapeDtypeStruct((N, W), jnp.float32),
    )(x, fc1_W, fc1_b.reshape(1, W))


def _msg_body(ea, xj, w0, b0, w1, b1, w2, b2, gm, out):
    f32 = jnp.float32
    bf16 = jnp.bfloat16
    t = jnp.maximum(jnp.dot(ea[...].astype(bf16), w0[...],
                            preferred_element_type=f32) + b0[...], 0.0)
    t = jnp.maximum(jnp.dot(t.astype(bf16), w1[...],
                            preferred_element_type=f32) + b1[...], 0.0)
    k2 = jnp.dot(t.astype(bf16), w2[...],
                 preferred_element_type=f32).astype(bf16) + b2[...]
    xe = jnp.concatenate([xj[...].astype(bf16)] * W, axis=1)
    out[...] = jnp.dot(k2 * xe, gm[...], preferred_element_type=f32)


def _tc_msg(edge_attr, xj, kW0, kb0, kW1, kb1, kW2, kb2, gm):
    rep = lambda i: (0, 0)
    return pl.pallas_call(
        _msg_body,
        grid=(E // BLK_E,),
        in_specs=[
            pl.BlockSpec((BLK_E, 6), lambda i: (i, 0)),
            pl.BlockSpec((BLK_E, W), lambda i: (i, 0)),
            pl.BlockSpec((6, 64), rep),
            pl.BlockSpec((1, 64), rep),
            pl.BlockSpec((64, 128), rep),
            pl.BlockSpec((1, 128), rep),
            pl.BlockSpec((128, 1024), rep),
            pl.BlockSpec((1, 1024), rep),
            pl.BlockSpec((1024, W), rep),
        ],
        out_specs=pl.BlockSpec((BLK_E, W), lambda i: (i, 0)),
        out_shape=jax.ShapeDtypeStruct((E, W), jnp.float32),
    )(edge_attr, xj, kW0, kb0.reshape(1, -1), kW1, kb1.reshape(1, -1),
      kW2, kb2.reshape(1, -1), gm)


def _comb_body(h_ref, s0, s1, d0, d1, w0, b0, w1, b1, w2, b2, gm,
               fw, fb, oh, oy):
    f32 = jnp.float32
    bf16 = jnp.bfloat16
    hb = h_ref[...]
    t = jnp.maximum(jnp.dot(hb.astype(bf16), w0[...],
                            preferred_element_type=f32) + b0[...], 0.0)
    t = jnp.maximum(jnp.dot(t.astype(bf16), w1[...],
                            preferred_element_type=f32) + b1[...], 0.0)
    bx = jnp.dot(t.astype(bf16), w2[...],
                 preferred_element_type=f32).astype(bf16) + b2[...]
    he = jnp.concatenate([hb.astype(bf16)] * W, axis=1)
    nl = jnp.dot(he * bx, gm[...], preferred_element_type=f32)
    deg = jnp.maximum(d0[...][0] + d1[...][0], 1.0)[:, 0:1]
    s = (s0[...][0] + s1[...][0]) / deg
    hn = 0.5 * s + 0.5 * (nl - hb) + hb
    oh[...] = hn
    oy[...] = jnp.dot(hn, fw[...], preferred_element_type=f32) + fb[...]


def _tc_combine(h, sp, dp, bW0, bb0, bW1, bb1, bW2, bb2, gm,
                fc2_W, fc2_b):
    rep = lambda i: (0, 0)
    blk = lambda i: (i, 0)
    return pl.pallas_call(
        _comb_body,
        grid=(N // BLK_N,),
        in_specs=[
            pl.BlockSpec((BLK_N, W), blk),
            pl.BlockSpec((1, BLK_N, W), lambda i: (0, i, 0)),
            pl.BlockSpec((1, BLK_N, W), lambda i: (1, i, 0)),
            pl.BlockSpec((1, BLK_N, W), lambda i: (0, i, 0)),
            pl.BlockSpec((1, BLK_N, W), lambda i: (1, i, 0)),
            pl.BlockSpec((W, 64), rep),
            pl.BlockSpec((1, 64), rep),
            pl.BlockSpec((64, 128), rep),
            pl.BlockSpec((1, 128), rep),
            pl.BlockSpec((128, 1024), rep),
            pl.BlockSpec((1, 1024), rep),
            pl.BlockSpec((1024, W), rep),
            pl.BlockSpec((W, 1), rep),
            pl.BlockSpec((1, 1), rep),
        ],
        out_specs=[
            pl.BlockSpec((BLK_N, W), blk),
            pl.BlockSpec((BLK_N, 1), blk),
        ],
        out_shape=[
            jax.ShapeDtypeStruct((N, W), jnp.float32),
            jax.ShapeDtypeStruct((N, 1), jnp.float32),
        ],
    )(h, sp, sp, dp, dp, bW0, bb0.reshape(1, -1),
      bW1, bb1.reshape(1, -1), bW2, bb2.reshape(1, -1), gm,
      fc2_W, fc2_b.reshape(1, 1))


def kernel(x, edge_index, edge_attr, fc1_W, fc1_b, kW0, kb0, kW1, kb1,
           kW2, kb2, bW0, bb0, bW1, bb1, bW2, bb2, fc2_W, fc2_b):
    src2d = edge_index[0].reshape(NCH, CHUNK)
    dst2d = edge_index[1].reshape(NCH, CHUNK)
    bf16 = jnp.bfloat16
    gm = jnp.asarray(G_MAT, dtype=bf16)
    kW0b, kW1b, kW2b = kW0.astype(bf16), kW1.astype(bf16), kW2.astype(bf16)
    bW0b, bW1b, bW2b = bW0.astype(bf16), bW1.astype(bf16), bW2.astype(bf16)
    kb2b, bb2b = kb2.astype(bf16), bb2.astype(bf16)

    h = _tc_fc1(x, fc1_W, fc1_b)
    dp = None
    out = None
    for d in range(DEPTH):
        xj = _sc_gather(h, src2d)
        msg = _tc_msg(edge_attr, xj, kW0b, kb0, kW1b, kb1, kW2b, kb2b, gm)
        if d == 0:
            sp, dp = _sc_scatter_deg(msg, dst2d)
        else:
            (sp,) = _sc_scatter(msg, dst2d)
        h, out = _tc_combine(h, sp, dp, bW0b, bb0, bW1b, bb1, bW2b, bb2b,
                             gm, fc2_W, fc2_b)
    return out


# speedup vs baseline: 1.0724x; 1.0198x over previous
import functools

import numpy as np
import jax
import jax.numpy as jnp
from jax import lax
from jax.experimental import pallas as pl
from jax.experimental.pallas import tpu as pltpu
from jax.experimental.pallas import tpu_sc as plsc

N = 10000
E = 160000
W = 32
DEPTH = 2

NC = 2
NS = 16
NWK = NC * NS
CHUNK = 128
NCH = E // CHUNK
CBASE = NCH // NWK
CEXTRA = NCH - CBASE * NWK
CPW = CBASE + 1
NBUF = 8
NP = 10240
TROWS = NP // NS

G_MAT = np.repeat(np.eye(W, dtype=np.float32), W, axis=0)

BLK_E = 6400
BLK_N = 2000

_MESH = plsc.VectorSubcoreMesh(core_axis_name="c", subcore_axis_name="s")
_SC_PARAMS = pltpu.CompilerParams(use_tc_tiling_on_sc=False)


def _stage_idx(idx_hbm, idx_v, wid):
    pltpu.sync_copy(idx_hbm.at[pl.ds(wid * CBASE, CBASE)],
                    idx_v.at[pl.ds(0, CBASE)])

    @pl.when(wid < CEXTRA)
    def _():
        pltpu.sync_copy(idx_hbm.at[CBASE * NWK + wid], idx_v.at[CBASE])


def _chunk_of(j, wid):
    return jnp.where(j < CBASE, wid * CBASE + j, CBASE * NWK + wid)


@functools.partial(
    pl.kernel,
    mesh=_MESH,
    compiler_params=_SC_PARAMS,
    out_type=jax.ShapeDtypeStruct((E, W), jnp.float32),
    scratch_types=[
        pltpu.VMEM((CPW, CHUNK), jnp.int32),
        [pltpu.VMEM((CHUNK, W), jnp.float32) for _ in range(NBUF)],
        [pltpu.SemaphoreType.DMA for _ in range(NBUF)],
    ],
)
def _sc_gather(h_hbm, idx_hbm, out_hbm, idx_v, rows, sems):
    wid = lax.axis_index("c") * NS + lax.axis_index("s")
    cnt = jnp.where(wid < CEXTRA, CBASE + 1, CBASE)
    _stage_idx(idx_hbm, idx_v, wid)

    def body(t, c):
        for b in range(NBUF):
            j = t * NBUF + b

            @pl.when(j < cnt)
            def _(b=b, j=j):
                pltpu.async_copy(h_hbm.at[idx_v.at[j]], rows[b], sems[b])

        for b in range(NBUF):
            j = t * NBUF + b

            @pl.when(j < cnt)
            def _(b=b, j=j):
                pltpu.make_async_copy(h_hbm.at[idx_v.at[j]], rows[b],
                                      sems[b]).wait()
                pltpu.sync_copy(
                    rows[b],
                    out_hbm.at[pl.ds(_chunk_of(j, wid) * CHUNK, CHUNK)])

        return c

    lax.fori_loop(0, -(-CPW // NBUF), body, 0)


def _make_sc_scatter(with_deg):
    out_types = [jax.ShapeDtypeStruct((NC, NP, W), jnp.float32)]
    scratch = [
        pltpu.VMEM((CPW, CHUNK), jnp.int32),
        [pltpu.VMEM((CHUNK, W), jnp.float32) for _ in range(NBUF)],
        [pltpu.SemaphoreType.DMA for _ in range(NBUF)],
        pltpu.VMEM((CHUNK, W), jnp.float32),
        pltpu.VMEM((TROWS, W), jnp.float32),
        pltpu.VMEM_SHARED((NP, W), jnp.float32),
    ]
    if with_deg:
        out_types.append(jax.ShapeDtypeStruct((NC, NP, W), jnp.float32))
        scratch += [
            pltpu.VMEM((CHUNK, W), jnp.float32),
            pltpu.VMEM_SHARED((NP, W), jnp.float32),
        ]

    def body_fn(msg_hbm, idx_hbm, *rest):
        if with_deg:
            (s_out, d_out, idx_v, rows, sems, zeros_v, stage_v, acc,
             ones_v, dacc) = rest
        else:
            (s_out, idx_v, rows, sems, zeros_v, stage_v, acc) = rest
        cid = lax.axis_index("c")
        sid = lax.axis_index("s")
        wid = cid * NS + sid
        cnt = jnp.where(wid < CEXTRA, CBASE + 1, CBASE)
        base = sid * TROWS

        _stage_idx(idx_hbm, idx_v, wid)

        z16 = jnp.zeros((16,), jnp.float32)
        o16 = jnp.ones((16,), jnp.float32)

        def fill(i, c):
            zeros_v[i, pl.ds(0, 16)] = z16
            zeros_v[i, pl.ds(16, 16)] = z16
            if with_deg:
                ones_v[i, pl.ds(0, 16)] = o16
                ones_v[i, pl.ds(16, 16)] = o16
            return c

        lax.fori_loop(0, CHUNK, fill, 0)

        def zero_acc(k, c):
            pltpu.sync_copy(zeros_v, acc.at[pl.ds(base + k * CHUNK, CHUNK)])
            if with_deg:
                pltpu.sync_copy(zeros_v, dacc.at[pl.ds(base + k * CHUNK, CHUNK)])
            return c

        lax.fori_loop(0, TROWS // CHUNK, zero_acc, 0)
        plsc.subcore_barrier()

        def body(t, c):
            for b in range(NBUF):
                j = t * NBUF + b

                @pl.when(j < cnt)
                def _(b=b, j=j):
                    pltpu.async_copy(
                        msg_hbm.at[pl.ds(_chunk_of(j, wid) * CHUNK, CHUNK)],
                        rows[b], sems[b])

            for b in range(NBUF):
                j = t * NBUF + b

                @pl.when(j < cnt)
                def _(b=b, j=j):
                    pltpu.make_async_copy(
                        msg_hbm.at[pl.ds(_chunk_of(j, wid) * CHUNK, CHUNK)],
                        rows[b], sems[b]).wait()
                    pltpu.sync_copy(rows[b], acc.at[idx_v.at[j]], add=True)
                    if with_deg:
                        pltpu.sync_copy(ones_v, dacc.at[idx_v.at[j]], add=True)

            return c

        lax.fori_loop(0, -(-CPW // NBUF), body, 0)
        plsc.subcore_barrier()

        pltpu.sync_copy(acc.at[pl.ds(base, TROWS)], stage_v)
        pltpu.sync_copy(stage_v, s_out.at[cid, pl.ds(base, TROWS)])
        if with_deg:
            pltpu.sync_copy(dacc.at[pl.ds(base, TROWS)], stage_v)
            pltpu.sync_copy(stage_v, d_out.at[cid, pl.ds(base, TROWS)])

    return pl.kernel(
        body_fn,
        mesh=_MESH,
        compiler_params=_SC_PARAMS,
        out_type=tuple(out_types),
        scratch_types=scratch,
    )


_sc_scatter_deg = _make_sc_scatter(True)
_sc_scatter = _make_sc_scatter(False)


def _fc1_body(x_ref, w_ref, b_ref, o_ref):
    o_ref[...] = x_ref[...] * w_ref[...] + b_ref[...]


def _tc_fc1(x, fc1_W, fc1_b):
    return pl.pallas_call(
        _fc1_body,
        grid=(N // BLK_N,),
        in_specs=[
            pl.BlockSpec((BLK_N, 1), lambda i: (i, 0)),
            pl.BlockSpec((1, W), lambda i: (0, 0)),
            pl.BlockSpec((1, W), lambda i: (0, 0)),
        ],
        out_specs=pl.BlockSpec((BLK_N, W), lambda i: (i, 0)),
        out_shape=jax.ShapeDtypeStruct((N, W), jnp.float32),
    )(x, fc1_W, fc1_b.reshape(1, W))


def _msg_body(ea, xj, w0, b0, w1, b1, w2, b2, gm, out):
    f32 = jnp.float32
    bf16 = jnp.bfloat16
    t = jnp.maximum(jnp.dot(ea[...].astype(bf16), w0[...],
                            preferred_element_type=f32) + b0[...], 0.0)
    t = jnp.maximum(jnp.dot(t.astype(bf16), w1[...],
                            preferred_element_type=f32) + b1[...], 0.0)
    k2 = jnp.dot(t.astype(bf16), w2[...],
                 preferred_element_type=f32).astype(bf16) + b2[...]
    xe = jnp.concatenate([xj[...].astype(bf16)] * W, axis=1)
    out[...] = jnp.dot(k2 * xe, gm[...], preferred_element_type=f32)


def _tc_msg(edge_attr, xj, kW0, kb0, kW1, kb1, kW2, kb2, gm):
    rep = lambda i: (0, 0)
    return pl.pallas_call(
        _msg_body,
        grid=(E // BLK_E,),
        in_specs=[
            pl.BlockSpec((BLK_E, 6), lambda i: (i, 0)),
            pl.BlockSpec((BLK_E, W), lambda i: (i, 0)),
            pl.BlockSpec((6, 64), rep),
            pl.BlockSpec((1, 64), rep),
            pl.BlockSpec((64, 128), rep),
            pl.BlockSpec((1, 128), rep),
            pl.BlockSpec((128, 1024), rep),
            pl.BlockSpec((1, 1024), rep),
            pl.BlockSpec((1024, W), rep),
        ],
        out_specs=pl.BlockSpec((BLK_E, W), lambda i: (i, 0)),
        out_shape=jax.ShapeDtypeStruct((E, W), jnp.float32),
    )(edge_attr, xj, kW0, kb0.reshape(1, -1), kW1, kb1.reshape(1, -1),
      kW2, kb2.reshape(1, -1), gm)


def _comb_body(h_ref, s0, s1, d0, d1, w0, b0, w1, b1, w2, b2, gm,
               fw, fb, oh, oy):
    f32 = jnp.float32
    bf16 = jnp.bfloat16
    hb = h_ref[...]
    t = jnp.maximum(jnp.dot(hb.astype(bf16), w0[...],
                            preferred_element_type=f32) + b0[...], 0.0)
    t = jnp.maximum(jnp.dot(t.astype(bf16), w1[...],
                            preferred_element_type=f32) + b1[...], 0.0)
    bx = jnp.dot(t.astype(bf16), w2[...],
                 preferred_element_type=f32).astype(bf16) + b2[...]
    he = jnp.concatenate([hb.astype(bf16)] * W, axis=1)
    nl = jnp.dot(he * bx, gm[...], preferred_element_type=f32)
    deg = jnp.maximum(d0[...][0] + d1[...][0], 1.0)[:, 0:1]
    s = (s0[...][0] + s1[...][0]) / deg
    hn = 0.5 * s + 0.5 * (nl - hb) + hb
    oh[...] = hn
    oy[...] = jnp.dot(hn, fw[...], preferred_element_type=f32) + fb[...]


def _tc_combine(h, sp, dp, bW0, bb0, bW1, bb1, bW2, bb2, gm,
                fc2_W, fc2_b):
    rep = lambda i: (0, 0)
    blk = lambda i: (i, 0)
    return pl.pallas_call(
        _comb_body,
        grid=(N // BLK_N,),
        in_specs=[
            pl.BlockSpec((BLK_N, W), blk),
            pl.BlockSpec((1, BLK_N, W), lambda i: (0, i, 0)),
            pl.BlockSpec((1, BLK_N, W), lambda i: (1, i, 0)),
            pl.BlockSpec((1, BLK_N, W), lambda i: (0, i, 0)),
            pl.BlockSpec((1, BLK_N, W), lambda i: (1, i, 0)),
            pl.BlockSpec((W, 64), rep),
            pl.BlockSpec((1, 64), rep),
            pl.BlockSpec((64, 128), rep),
            pl.BlockSpec((1, 128), rep),
            pl.BlockSpec((128, 1024), rep),
            pl.BlockSpec((1, 1024), rep),
            pl.BlockSpec((1024, W), rep),
            pl.BlockSpec((W, 1), rep),
            pl.BlockSpec((1, 1), rep),
        ],
        out_specs=[
            pl.BlockSpec((BLK_N, W), blk),
            pl.BlockSpec((BLK_N, 1), blk),
        ],
        out_shape=[
            jax.ShapeDtypeStruct((N, W), jnp.float32),
            jax.ShapeDtypeStruct((N, 1), jnp.float32),
        ],
    )(h, sp, sp, dp, dp, bW0, bb0.reshape(1, -1),
      bW1, bb1.reshape(1, -1), bW2, bb2.reshape(1, -1), gm,
      fc2_W, fc2_b.reshape(1, 1))


def kernel(x, edge_index, edge_attr, fc1_W, fc1_b, kW0, kb0, kW1, kb1,
           kW2, kb2, bW0, bb0, bW1, bb1, bW2, bb2, fc2_W, fc2_b):
    src2d = edge_index[0].reshape(NCH, CHUNK)
    dst2d = edge_index[1].reshape(NCH, CHUNK)
    bf16 = jnp.bfloat16
    gm = jnp.asarray(G_MAT, dtype=bf16)
    kW0b, kW1b, kW2b = kW0.astype(bf16), kW1.astype(bf16), kW2.astype(bf16)
    bW0b, bW1b, bW2b = bW0.astype(bf16), bW1.astype(bf16), bW2.astype(bf16)
    kb2b, bb2b = kb2.astype(bf16), bb2.astype(bf16)

    h = _tc_fc1(x, fc1_W, fc1_b)
    dp = None
    out = None
    for d in range(DEPTH):
        xj = _sc_gather(h, src2d)
        msg = _tc_msg(edge_attr, xj, kW0b, kb0, kW1b, kb1, kW2b, kb2b, gm)
        if d == 0:
            sp, dp = _sc_scatter_deg(msg, dst2d)
        else:
            (sp,) = _sc_scatter(msg, dst2d)
        h, out = _tc_combine(h, sp, dp, bW0b, bb0, bW1b, bb1, bW2b, bb2b,
                             gm, fc2_W, fc2_b)
    return out


# speedup vs baseline: 1.0914x; 1.0178x over previous
import functools

import numpy as np
import jax
import jax.numpy as jnp
from jax import lax
from jax.experimental import pallas as pl
from jax.experimental.pallas import tpu as pltpu
from jax.experimental.pallas import tpu_sc as plsc

N = 10000
E = 160000
W = 32
DEPTH = 2

NC = 2
NS = 16
NWK = NC * NS
CHUNK = 128
NCH = E // CHUNK
CBASE = NCH // NWK
CEXTRA = NCH - CBASE * NWK
CPW = CBASE + 1
NBUF = 8
NP = 10240
TROWS = NP // NS

G_MAT = np.repeat(np.eye(W, dtype=np.float32), W, axis=0)

BLK_E = 6400
BLK_N = 2000

_MESH = plsc.VectorSubcoreMesh(core_axis_name="c", subcore_axis_name="s")
_SC_PARAMS = pltpu.CompilerParams(use_tc_tiling_on_sc=False)


def _stage_idx(idx_hbm, idx_v, wid):
    pltpu.sync_copy(idx_hbm.at[pl.ds(wid * CBASE, CBASE)],
                    idx_v.at[pl.ds(0, CBASE)])

    @pl.when(wid < CEXTRA)
    def _():
        pltpu.sync_copy(idx_hbm.at[CBASE * NWK + wid], idx_v.at[CBASE])


def _chunk_of(j, wid):
    return jnp.where(j < CBASE, wid * CBASE + j, CBASE * NWK + wid)


def _make_sc_gather(with_deg):
    out_types = [jax.ShapeDtypeStruct((E, W), jnp.float32)]
    scratch = [
        pltpu.VMEM((CPW, CHUNK), jnp.int32),
        [pltpu.VMEM((CHUNK, W), jnp.float32) for _ in range(NBUF)],
        [pltpu.SemaphoreType.DMA for _ in range(NBUF)],
    ]
    if with_deg:
        out_types.append(jax.ShapeDtypeStruct((NC, NP, W), jnp.float32))
        scratch += [
            pltpu.VMEM((CPW, CHUNK), jnp.int32),
            pltpu.VMEM((CHUNK, W), jnp.float32),
            pltpu.VMEM((CHUNK, W), jnp.float32),
            pltpu.VMEM((TROWS, W), jnp.float32),
            pltpu.VMEM_SHARED((NP, W), jnp.float32),
        ]

    def body_fn(h_hbm, idx_hbm, *rest):
        if with_deg:
            (didx_hbm, out_hbm, d_out, idx_v, rows, sems,
             didx_v, ones_v, zeros_v, stage_v, dacc) = rest
        else:
            (out_hbm, idx_v, rows, sems) = rest
        cid = lax.axis_index("c")
        sid = lax.axis_index("s")
        wid = cid * NS + sid
        cnt = jnp.where(wid < CEXTRA, CBASE + 1, CBASE)
        base = sid * TROWS
        _stage_idx(idx_hbm, idx_v, wid)

        if with_deg:
            _stage_idx(didx_hbm, didx_v, wid)
            z16 = jnp.zeros((16,), jnp.float32)
            o16 = jnp.ones((16,), jnp.float32)

            def fill(i, c):
                zeros_v[i, pl.ds(0, 16)] = z16
                zeros_v[i, pl.ds(16, 16)] = z16
                ones_v[i, pl.ds(0, 16)] = o16
                ones_v[i, pl.ds(16, 16)] = o16
                return c

            lax.fori_loop(0, CHUNK, fill, 0)

            def zero_acc(k, c):
                pltpu.sync_copy(zeros_v,
                                dacc.at[pl.ds(base + k * CHUNK, CHUNK)])
                return c

            lax.fori_loop(0, TROWS // CHUNK, zero_acc, 0)
            plsc.subcore_barrier()

        def body(t, c):
            for b in range(NBUF):
                j = t * NBUF + b

                @pl.when(j < cnt)
                def _(b=b, j=j):
                    pltpu.async_copy(h_hbm.at[idx_v.at[j]], rows[b], sems[b])

            for b in range(NBUF):
                j = t * NBUF + b

                @pl.when(j < cnt)
                def _(b=b, j=j):
                    if with_deg:
                        pltpu.sync_copy(ones_v, dacc.at[didx_v.at[j]],
                                        add=True)
                    pltpu.make_async_copy(h_hbm.at[idx_v.at[j]], rows[b],
                                          sems[b]).wait()
                    pltpu.sync_copy(
                        rows[b],
                        out_hbm.at[pl.ds(_chunk_of(j, wid) * CHUNK, CHUNK)])

            return c

        lax.fori_loop(0, -(-CPW // NBUF), body, 0)

        if with_deg:
            plsc.subcore_barrier()
            pltpu.sync_copy(dacc.at[pl.ds(base, TROWS)], stage_v)
            pltpu.sync_copy(stage_v, d_out.at[cid, pl.ds(base, TROWS)])

    return pl.kernel(
        body_fn,
        mesh=_MESH,
        compiler_params=_SC_PARAMS,
        out_type=tuple(out_types),
        scratch_types=scratch,
    )


_sc_gather_deg = _make_sc_gather(True)
_sc_gather = _make_sc_gather(False)


def _make_sc_scatter(with_deg):
    out_types = [jax.ShapeDtypeStruct((NC, NP, W), jnp.float32)]
    scratch = [
        pltpu.VMEM((CPW, CHUNK), jnp.int32),
        [pltpu.VMEM((CHUNK, W), jnp.float32) for _ in range(NBUF)],
        [pltpu.SemaphoreType.DMA for _ in range(NBUF)],
        pltpu.VMEM((CHUNK, W), jnp.float32),
        pltpu.VMEM((TROWS, W), jnp.float32),
        pltpu.VMEM_SHARED((NP, W), jnp.float32),
    ]
    if with_deg:
        out_types.append(jax.ShapeDtypeStruct((NC, NP, W), jnp.float32))
        scratch += [
            pltpu.VMEM((CHUNK, W), jnp.float32),
            pltpu.VMEM_SHARED((NP, W), jnp.float32),
        ]

    def body_fn(msg_hbm, idx_hbm, *rest):
        if with_deg:
            (s_out, d_out, idx_v, rows, sems, zeros_v, stage_v, acc,
             ones_v, dacc) = rest
        else:
            (s_out, idx_v, rows, sems, zeros_v, stage_v, acc) = rest
        cid = lax.axis_index("c")
        sid = lax.axis_index("s")
        wid = cid * NS + sid
        cnt = jnp.where(wid < CEXTRA, CBASE + 1, CBASE)
        base = sid * TROWS

        _stage_idx(idx_hbm, idx_v, wid)

        z16 = jnp.zeros((16,), jnp.float32)
        o16 = jnp.ones((16,), jnp.float32)

        def fill(i, c):
            zeros_v[i, pl.ds(0, 16)] = z16
            zeros_v[i, pl.ds(16, 16)] = z16
            if with_deg:
                ones_v[i, pl.ds(0, 16)] = o16
                ones_v[i, pl.ds(16, 16)] = o16
            return c

        lax.fori_loop(0, CHUNK, fill, 0)

        def zero_acc(k, c):
            pltpu.sync_copy(zeros_v, acc.at[pl.ds(base + k * CHUNK, CHUNK)])
            if with_deg:
                pltpu.sync_copy(zeros_v, dacc.at[pl.ds(base + k * CHUNK, CHUNK)])
            return c

        lax.fori_loop(0, TROWS // CHUNK, zero_acc, 0)
        plsc.subcore_barrier()

        def body(t, c):
            for b in range(NBUF):
                j = t * NBUF + b

                @pl.when(j < cnt)
                def _(b=b, j=j):
                    pltpu.async_copy(
                        msg_hbm.at[pl.ds(_chunk_of(j, wid) * CHUNK, CHUNK)],
                        rows[b], sems[b])

            for b in range(NBUF):
                j = t * NBUF + b

                @pl.when(j < cnt)
                def _(b=b, j=j):
                    pltpu.make_async_copy(
                        msg_hbm.at[pl.ds(_chunk_of(j, wid) * CHUNK, CHUNK)],
                        rows[b], sems[b]).wait()
                    pltpu.sync_copy(rows[b], acc.at[idx_v.at[j]], add=True)
                    if with_deg:
                        pltpu.sync_copy(ones_v, dacc.at[idx_v.at[j]], add=True)

            return c

        lax.fori_loop(0, -(-CPW // NBUF), body, 0)
        plsc.subcore_barrier()

        pltpu.sync_copy(acc.at[pl.ds(base, TROWS)], stage_v)
        pltpu.sync_copy(stage_v, s_out.at[cid, pl.ds(base, TROWS)])
        if with_deg:
            pltpu.sync_copy(dacc.at[pl.ds(base, TROWS)], stage_v)
            pltpu.sync_copy(stage_v, d_out.at[cid, pl.ds(base, TROWS)])

    return pl.kernel(
        body_fn,
        mesh=_MESH,
        compiler_params=_SC_PARAMS,
        out_type=tuple(out_types),
        scratch_types=scratch,
    )


_sc_scatter = _make_sc_scatter(False)


def _fc1_body(x_ref, w_ref, b_ref, o_ref):
    o_ref[...] = x_ref[...] * w_ref[...] + b_ref[...]


def _tc_fc1(x, fc1_W, fc1_b):
    return pl.pallas_call(
        _fc1_body,
        grid=(N // BLK_N,),
        in_specs=[
            pl.BlockSpec((BLK_N, 1), lambda i: (i, 0)),
            pl.BlockSpec((1, W), lambda i: (0, 0)),
            pl.BlockSpec((1, W), lambda i: (0, 0)),
        ],
        out_specs=pl.BlockSpec((BLK_N, W), lambda i: (i, 0)),
        out_shape=jax.ShapeDtypeStruct((N, W), jnp.float32),
    )(x, fc1_W, fc1_b.reshape(1, W))


def _msg_body(ea, xj, w0, b0, w1, b1, w2, b2, gm, out):
    f32 = jnp.float32
    bf16 = jnp.bfloat16
    t = jnp.maximum(jnp.dot(ea[...].astype(bf16), w0[...],
                            preferred_element_type=f32) + b0[...], 0.0)
    t = jnp.maximum(jnp.dot(t.astype(bf16), w1[...],
                            preferred_element_type=f32) + b1[...], 0.0)
    k2 = jnp.dot(t.astype(bf16), w2[...],
                 preferred_element_type=f32).astype(bf16) + b2[...]
    xe = jnp.concatenate([xj[...].astype(bf16)] * W, axis=1)
    out[...] = jnp.dot(k2 * xe, gm[...], preferred_element_type=f32)


def _tc_msg(edge_attr, xj, kW0, kb0, kW1, kb1, kW2, kb2, gm):
    rep = lambda i: (0, 0)
    return pl.pallas_call(
        _msg_body,
        grid=(E // BLK_E,),
        in_specs=[
            pl.BlockSpec((BLK_E, 6), lambda i: (i, 0)),
            pl.BlockSpec((BLK_E, W), lambda i: (i, 0)),
            pl.BlockSpec((6, 64), rep),
            pl.BlockSpec((1, 64), rep),
            pl.BlockSpec((64, 128), rep),
            pl.BlockSpec((1, 128), rep),
            pl.BlockSpec((128, 1024), rep),
            pl.BlockSpec((1, 1024), rep),
            pl.BlockSpec((1024, W), rep),
        ],
        out_specs=pl.BlockSpec((BLK_E, W), lambda i: (i, 0)),
        out_shape=jax.ShapeDtypeStruct((E, W), jnp.float32),
    )(edge_attr, xj, kW0, kb0.reshape(1, -1), kW1, kb1.reshape(1, -1),
      kW2, kb2.reshape(1, -1), gm)


def _comb_body(h_ref, s0, s1, d0, d1, w0, b0, w1, b1, w2, b2, gm,
               fw, fb, oh, oy):
    f32 = jnp.float32
    bf16 = jnp.bfloat16
    hb = h_ref[...]
    t = jnp.maximum(jnp.dot(hb.astype(bf16), w0[...],
                            preferred_element_type=f32) + b0[...], 0.0)
    t = jnp.maximum(jnp.dot(t.astype(bf16), w1[...],
                            preferred_element_type=f32) + b1[...], 0.0)
    bx = jnp.dot(t.astype(bf16), w2[...],
                 preferred_element_type=f32).astype(bf16) + b2[...]
    he = jnp.concatenate([hb.astype(bf16)] * W, axis=1)
    nl = jnp.dot(he * bx, gm[...], preferred_element_type=f32)
    deg = jnp.maximum(d0[...][0] + d1[...][0], 1.0)[:, 0:1]
    s = (s0[...][0] + s1[...][0]) / deg
    hn = 0.5 * s + 0.5 * (nl - hb) + hb
    oh[...] = hn
    oy[...] = jnp.dot(hn, fw[...], preferred_element_type=f32) + fb[...]


def _tc_combine(h, sp, dp, bW0, bb0, bW1, bb1, bW2, bb2, gm,
                fc2_W, fc2_b):
    rep = lambda i: (0, 0)
    blk = lambda i: (i, 0)
    return pl.pallas_call(
        _comb_body,
        grid=(N // BLK_N,),
        in_specs=[
            pl.BlockSpec((BLK_N, W), blk),
            pl.BlockSpec((1, BLK_N, W), lambda i: (0, i, 0)),
            pl.BlockSpec((1, BLK_N, W), lambda i: (1, i, 0)),
            pl.BlockSpec((1, BLK_N, W), lambda i: (0, i, 0)),
            pl.BlockSpec((1, BLK_N, W), lambda i: (1, i, 0)),
            pl.BlockSpec((W, 64), rep),
            pl.BlockSpec((1, 64), rep),
            pl.BlockSpec((64, 128), rep),
            pl.BlockSpec((1, 128), rep),
            pl.BlockSpec((128, 1024), rep),
            pl.BlockSpec((1, 1024), rep),
            pl.BlockSpec((1024, W), rep),
            pl.BlockSpec((W, 1), rep),
            pl.BlockSpec((1, 1), rep),
        ],
        out_specs=[
            pl.BlockSpec((BLK_N, W), blk),
            pl.BlockSpec((BLK_N, 1), blk),
        ],
        out_shape=[
            jax.ShapeDtypeStruct((N, W), jnp.float32),
            jax.ShapeDtypeStruct((N, 1), jnp.float32),
        ],
    )(h, sp, sp, dp, dp, bW0, bb0.reshape(1, -1),
      bW1, bb1.reshape(1, -1), bW2, bb2.reshape(1, -1), gm,
      fc2_W, fc2_b.reshape(1, 1))


def kernel(x, edge_index, edge_attr, fc1_W, fc1_b, kW0, kb0, kW1, kb1,
           kW2, kb2, bW0, bb0, bW1, bb1, bW2, bb2, fc2_W, fc2_b):
    src2d = edge_index[0].reshape(NCH, CHUNK)
    dst2d = edge_index[1].reshape(NCH, CHUNK)
    bf16 = jnp.bfloat16
    gm = jnp.asarray(G_MAT, dtype=bf16)
    kW0b, kW1b, kW2b = kW0.astype(bf16), kW1.astype(bf16), kW2.astype(bf16)
    bW0b, bW1b, bW2b = bW0.astype(bf16), bW1.astype(bf16), bW2.astype(bf16)
    kb2b, bb2b = kb2.astype(bf16), bb2.astype(bf16)

    h = _tc_fc1(x, fc1_W, fc1_b)
    dp = None
    out = None
    for d in range(DEPTH):
        if d == 0:
            xj, dp = _sc_gather_deg(h, src2d, dst2d)
        else:
            (xj,) = _sc_gather(h, src2d)
        msg = _tc_msg(edge_attr, xj, kW0b, kb0, kW1b, kb1, kW2b, kb2b, gm)
        (sp,) = _sc_scatter(msg, dst2d)
        h, out = _tc_combine(h, sp, dp, bW0b, bb0, bW1b, bb1, bW2b, bb2b,
                             gm, fc2_W, fc2_b)
    return out
